# baseline (speedup 1.0000x reference)
import functools

import jax
import jax.numpy as jnp
from jax import lax
from jax.experimental import pallas as pl
from jax.experimental.pallas import tpu as pltpu

VOCAB = 8192
BATCH = 4096
D = VOCAB
Q = 16


def _tc_body(idx_ref, table_ref, out_ref, sems):
    def make(i, slot):
        return pltpu.make_async_copy(
            table_ref.at[pl.ds(idx_ref[i], 1)],
            out_ref.at[pl.ds(i, 1)],
            sems.at[slot],
        )

    def body(i, _):
        slot = lax.rem(i, Q)

        @pl.when(i >= Q)
        def _():
            make(i - Q, slot).wait()

        make(i, slot).start()
        return 0

    lax.fori_loop(0, BATCH, body, 0)

    def drain(i, _):
        make(i, lax.rem(i, Q)).wait()
        return 0

    lax.fori_loop(BATCH - Q, BATCH, drain, 0)


@jax.jit
def _lookup(idx, table):
    return pl.pallas_call(
        _tc_body,
        out_shape=jax.ShapeDtypeStruct((BATCH, D), jnp.float32),
        in_specs=[
            pl.BlockSpec(memory_space=pltpu.SMEM),
            pl.BlockSpec(memory_space=pl.ANY),
        ],
        out_specs=pl.BlockSpec(memory_space=pl.ANY),
        scratch_shapes=[pltpu.SemaphoreType.DMA((Q,))],
    )(idx, table)


def kernel(x, table):
    last = x[:, -1].astype(jnp.int32)
    return _lookup(last, table)


# speedup vs baseline: 6.6715x; 6.6715x over previous
import functools

import jax
import jax.numpy as jnp
from jax import lax
from jax.experimental import pallas as pl
from jax.experimental.pallas import tpu as pltpu

VOCAB = 8192
BATCH = 4096
D = VOCAB
RPS = 8


def _tc_body(idx_ref, *refs):
    in_refs = refs[:RPS]
    out_ref = refs[RPS]
    for j in range(RPS):
        out_ref[pl.ds(j, 1)] = in_refs[j][...]


@jax.jit
def _lookup(idx, table3):
    grid_spec = pltpu.PrefetchScalarGridSpec(
        num_scalar_prefetch=1,
        grid=(BATCH // RPS,),
        in_specs=[
            pl.BlockSpec((1, 8, D // 8), functools.partial(
                lambda j, i, idx_ref: (idx_ref[RPS * i + j], 0, 0), j))
            for j in range(RPS)
        ],
        out_specs=pl.BlockSpec((RPS, 8, D // 8), lambda i, idx_ref: (i, 0, 0)),
    )
    return pl.pallas_call(
        _tc_body,
        grid_spec=grid_spec,
        out_shape=jax.ShapeDtypeStruct((BATCH, 8, D // 8), jnp.float32),
    )(idx, *([table3] * RPS))


def kernel(x, table):
    last = x[:, -1].astype(jnp.int32)
    table3 = table.reshape(VOCAB, 8, D // 8)
    return _lookup(last, table3).reshape(BATCH, D)


# speedup vs baseline: 14.4494x; 2.1658x over previous
import functools

import jax
import jax.numpy as jnp
from jax import lax
from jax.experimental import pallas as pl
from jax.experimental.pallas import tpu as pltpu
from jax.experimental.pallas import tpu_sc as plsc

VOCAB = 8192
BATCH = 4096
D = VOCAB

NUM_CORES = 2
NUM_SUBCORES = 16
NW = NUM_CORES * NUM_SUBCORES
B_PER_W = BATCH // NW
CHUNK = 4
N_CHUNKS = B_PER_W // CHUNK


def _gather_body(idx_hbm, table_hbm, out3_hbm, idx_v, idx_g, rows_v, gsem):
    wid = lax.axis_index("s") * NUM_CORES + lax.axis_index("c")
    base = wid * B_PER_W
    cid0 = wid * N_CHUNKS

    pltpu.sync_copy(idx_hbm.at[pl.ds(base, B_PER_W)], idx_v)
    for r in range(B_PER_W // 16):
        idx_g[r] = idx_v[pl.ds(r * 16, 16)]

    for i in range(N_CHUNKS):
        idx_sl = idx_g.at[i // 4, pl.ds(CHUNK * (i % 4), CHUNK)]
        pltpu.async_copy(table_hbm.at[idx_sl], rows_v, gsem).wait()
        pltpu.sync_copy(rows_v, out3_hbm.at[cid0 + i])


@jax.jit
def _lookup(idx, table):
    mesh = plsc.VectorSubcoreMesh(core_axis_name="c", subcore_axis_name="s")
    kfn = functools.partial(
        pl.kernel,
        mesh=mesh,
        out_type=jax.ShapeDtypeStruct((NW * N_CHUNKS, CHUNK, D), jnp.float32),
        scratch_types=[
            pltpu.VMEM((B_PER_W,), jnp.int32),
            pltpu.VMEM((B_PER_W // 16, 16), jnp.int32),
            pltpu.VMEM((CHUNK, D), jnp.float32),
            pltpu.SemaphoreType.DMA,
        ],
    )(_gather_body)
    return kfn(idx, table)


def kernel(x, table):
    last = x[:, -1].astype(jnp.int32)
    return _lookup(last, table).reshape(BATCH, D)


# speedup vs baseline: 32.6823x; 2.2619x over previous
import functools

import jax
import jax.numpy as jnp
from jax import lax
from jax.experimental import pallas as pl
from jax.experimental.pallas import tpu as pltpu
from jax.experimental.pallas import tpu_sc as plsc

VOCAB = 8192
BATCH = 4096
D = VOCAB

NUM_CORES = 2
NUM_SUBCORES = 16
NW = NUM_CORES * NUM_SUBCORES
B_PER_W = BATCH // NW
CHUNK = 8
N_CHUNKS = B_PER_W // CHUNK


def _gather_body(idx_hbm, table_hbm, out_hbm, idx_v, rows_v, gsem):
    wid = lax.axis_index("s") * NUM_CORES + lax.axis_index("c")
    base = wid * B_PER_W

    pltpu.sync_copy(idx_hbm.at[pl.ds(base, B_PER_W)], idx_v)

    for i in range(N_CHUNKS):
        idx_sl = idx_v.at[pl.ds(i * CHUNK, CHUNK)]
        pltpu.async_copy(table_hbm.at[idx_sl], rows_v, gsem).wait()
        pltpu.sync_copy(rows_v, out_hbm.at[pl.ds(base + i * CHUNK, CHUNK)])


@jax.jit
def _lookup(idx, table):
    mesh = plsc.VectorSubcoreMesh(core_axis_name="c", subcore_axis_name="s")
    kfn = functools.partial(
        pl.kernel,
        mesh=mesh,
        out_type=jax.ShapeDtypeStruct((BATCH, D), jnp.float32),
        scratch_types=[
            pltpu.VMEM((B_PER_W,), jnp.int32),
            pltpu.VMEM((CHUNK, D), jnp.float32),
            pltpu.SemaphoreType.DMA,
        ],
    )(_gather_body)
    return kfn(idx, table)


def kernel(x, table):
    last = x[:, -1].astype(jnp.int32)
    return _lookup(last, table)
